# Initial kernel scaffold; baseline (speedup 1.0000x reference)
import jax
import jax.numpy as jnp
from jax.experimental import pallas as pl


def kernel(x, input_weight):
    raise NotImplementedError("write your pallas kernel here")


# speedup vs baseline: 5.9813x; 5.9813x over previous
import functools

import jax
import jax.numpy as jnp
from jax import lax
from jax.experimental import pallas as pl
from jax.experimental.pallas import tpu as pltpu
from jax.experimental.pallas import tpu_sc as plsc

EMB = 64
B_TOTAL = 16384 * 50
NUM_WORKERS = 32
PER_W = B_TOTAL // NUM_WORKERS
CHUNK = 800
NCHUNK = PER_W // CHUNK


def _emb_body(x_hbm, tab_hbm, out_hbm, idx_v, rows_v, sem):
    wid = lax.axis_index("s") * 2 + lax.axis_index("c")
    base = wid * PER_W

    def step(i, carry):
        off = base + i * CHUNK
        pltpu.sync_copy(x_hbm.at[pl.ds(off, CHUNK)], idx_v)
        pltpu.async_copy(tab_hbm.at[idx_v], rows_v, sem).wait()
        pltpu.sync_copy(rows_v, out_hbm.at[pl.ds(off, CHUNK)])
        return carry

    lax.fori_loop(0, NCHUNK, step, 0)


_emb = functools.partial(
    pl.kernel,
    out_type=jax.ShapeDtypeStruct((B_TOTAL, EMB), jnp.float32),
    mesh=plsc.VectorSubcoreMesh(core_axis_name="c", subcore_axis_name="s"),
    scratch_types=[
        pltpu.VMEM((CHUNK,), jnp.int32),
        pltpu.VMEM((CHUNK, EMB), jnp.float32),
        pltpu.SemaphoreType.DMA,
    ],
    compiler_params=pltpu.CompilerParams(use_tc_tiling_on_sc=False),
)(_emb_body)


def kernel(x, input_weight):
    flat = x.reshape(-1).astype(jnp.int32)
    out = _emb(flat, input_weight)
    return out.reshape(x.shape + (EMB,))


# speedup vs baseline: 6.2313x; 1.0418x over previous
import functools

import jax
import jax.numpy as jnp
from jax import lax
from jax.experimental import pallas as pl
from jax.experimental.pallas import tpu as pltpu
from jax.experimental.pallas import tpu_sc as plsc

EMB = 64
B_TOTAL = 16384 * 50
NUM_WORKERS = 32
PER_W = B_TOTAL // NUM_WORKERS
CHUNK = 800
NCHUNK = PER_W // CHUNK


def _emb_body(x_hbm, tab_hbm, out_hbm,
              idx_all, rows0, rows1, gsem0, gsem1, wsem0, wsem1):
    wid = lax.axis_index("s") * 2 + lax.axis_index("c")
    base = wid * PER_W
    rows = (rows0, rows1)
    gsem = (gsem0, gsem1)
    wsem = (wsem0, wsem1)

    pltpu.sync_copy(x_hbm.at[pl.ds(base, PER_W)], idx_all)

    def gather(i, b):
        return pltpu.async_copy(
            tab_hbm.at[idx_all.at[pl.ds(i * CHUNK, CHUNK)]], rows[b], gsem[b])

    def wait_gather(i, b):
        pltpu.make_async_copy(
            tab_hbm.at[idx_all.at[pl.ds(i * CHUNK, CHUNK)]], rows[b],
            gsem[b]).wait()

    def writeout(i, b):
        return pltpu.async_copy(
            rows[b], out_hbm.at[pl.ds(base + i * CHUNK, CHUNK)], wsem[b])

    def wait_writeout(i, b):
        pltpu.make_async_copy(
            rows[b], out_hbm.at[pl.ds(base + i * CHUNK, CHUNK)],
            wsem[b]).wait()

    gather(0, 0)
    gather(1, 1)

    def outer(g, carry):
        for b in range(2):
            i = 2 * g + b
            wait_gather(i, b)
            writeout(i, b)
            wait_writeout(i, b)
            gather(i + 2, b)
        return carry

    lax.fori_loop(0, NCHUNK // 2 - 1, outer, 0)

    for b in range(2):
        i = NCHUNK - 2 + b
        wait_gather(i, b)
        writeout(i, b)
    for b in range(2):
        wait_writeout(NCHUNK - 2 + b, b)


_emb = functools.partial(
    pl.kernel,
    out_type=jax.ShapeDtypeStruct((B_TOTAL, EMB), jnp.float32),
    mesh=plsc.VectorSubcoreMesh(core_axis_name="c", subcore_axis_name="s"),
    scratch_types=[
        pltpu.VMEM((PER_W,), jnp.int32),
        pltpu.VMEM((CHUNK, EMB), jnp.float32),
        pltpu.VMEM((CHUNK, EMB), jnp.float32),
        pltpu.SemaphoreType.DMA,
        pltpu.SemaphoreType.DMA,
        pltpu.SemaphoreType.DMA,
        pltpu.SemaphoreType.DMA,
    ],
    compiler_params=pltpu.CompilerParams(use_tc_tiling_on_sc=False),
)(_emb_body)


def kernel(x, input_weight):
    flat = x.reshape(-1).astype(jnp.int32)
    out = _emb(flat, input_weight)
    return out.reshape(x.shape + (EMB,))
